# baseline (speedup 1.0000x reference)
import functools

import jax
import jax.numpy as jnp
from jax.experimental import pallas as pl
from jax.experimental.pallas import tpu as pltpu
from jax.experimental.pallas import tpu_sc as plsc

_T, _D, _G, _H, _E = 2048, 768, 256, 1536, 8
_B = 512
_NBLK = (2 * _T) // _B + _E
_NROWS = _NBLK * _B
_W = 64


def _gate_kernel(x_ref, gw1_ref, gb1_ref, gw2_ref, gb2_ref,
                 w12T_ref, posT_ref, bexp_ref, nact_ref, aux_ref):
    x = x_ref[...]
    gh = jnp.maximum(
        jnp.dot(x, gw1_ref[...], preferred_element_type=jnp.float32)
        + gb1_ref[...], 0.0)
    logits = (jnp.dot(gh, gw2_ref[...], preferred_element_type=jnp.float32)
              + gb2_ref[...])
    lt = logits.T
    m = jnp.max(lt, axis=0, keepdims=True)
    ex = jnp.exp(lt - m)
    p = ex / jnp.sum(ex, axis=0, keepdims=True)

    sub = jax.lax.broadcasted_iota(jnp.int32, (_E, _T), 0)
    m1 = jnp.max(p, axis=0, keepdims=True)
    i1 = jnp.min(jnp.where(p == m1, sub, _E), axis=0, keepdims=True)
    pm = jnp.where(sub == i1, -1.0, p)
    m2 = jnp.max(pm, axis=0, keepdims=True)
    i2 = jnp.min(jnp.where(pm == m2, sub, _E), axis=0, keepdims=True)

    denom = m1 + m2 + 1e-9
    w12T_ref[...] = jnp.concatenate([m1 / denom, m2 / denom], axis=0)

    oh1 = jnp.where(sub == i1, 1.0, 0.0)
    oh2 = jnp.where(sub == i2, 1.0, 0.0)
    c = oh1 + oh2
    cnt = jnp.sum(c, axis=1, keepdims=True)
    load = cnt / (_T + 1e-9)
    lbm = jnp.mean(load)
    lbl = jnp.sum((load - lbm) ** 2) / (_E - 1)
    ent = -jnp.sum(p * jnp.log(p + 1e-9)) / _T
    aux_ref[...] = jnp.reshape(lbl + ent, (1, 1))

    _CH = 128
    _NC = _T // _CH
    r_i = jax.lax.broadcasted_iota(jnp.int32, (_CH, _CH), 0)
    c_i = jax.lax.broadcasted_iota(jnp.int32, (_CH, _CH), 1)
    ustri = jnp.where(r_i < c_i, 1.0, 0.0)
    parts = []
    sums = []
    for k in range(_NC):
        ck = c[:, k * _CH:(k + 1) * _CH]
        parts.append(jnp.dot(ck, ustri, preferred_element_type=jnp.float32))
        sums.append(jnp.sum(ck, axis=1, keepdims=True))
    s = jnp.concatenate(sums, axis=1)
    kr_i = jax.lax.broadcasted_iota(jnp.int32, (_NC, _NC), 0)
    kc_i = jax.lax.broadcasted_iota(jnp.int32, (_NC, _NC), 1)
    kustri = jnp.where(kr_i < kc_i, 1.0, 0.0)
    pref = jnp.dot(s, kustri, preferred_element_type=jnp.float32)
    rank = jnp.concatenate(
        [parts[k] + pref[:, k:k + 1] for k in range(_NC)], axis=1)

    nblk_e = jnp.floor((cnt + (_B - 1)) / _B)
    e_r = jax.lax.broadcasted_iota(jnp.int32, (_E, _E), 0)
    e_c = jax.lax.broadcasted_iota(jnp.int32, (_E, _E), 1)
    ltri_inc = jnp.where(e_c <= e_r, 1.0, 0.0)
    ends = jnp.dot(ltri_inc, nblk_e,
                   preferred_element_type=jnp.float32)
    starts = ends - nblk_e
    rowoff = starts * _B

    posm = rowoff + rank
    pos1 = jnp.sum(jnp.where(sub == i1, posm, 0.0), axis=0, keepdims=True)
    pos2 = jnp.sum(jnp.where(sub == i2, posm, 0.0), axis=0, keepdims=True)
    posT_ref[...] = jnp.concatenate([pos1, pos2], axis=0).astype(jnp.int32)

    b_iota = jax.lax.broadcasted_iota(jnp.int32, (1, _NBLK), 1).astype(
        jnp.float32)
    bexp = jnp.zeros((1, _NBLK), jnp.float32)
    for e in range(_E):
        bexp = bexp + jnp.where(b_iota >= ends[e:e + 1, 0:1], 1.0, 0.0)
    bexp_ref[...] = jnp.minimum(bexp, _E - 1).astype(jnp.int32)
    nact_ref[...] = jnp.sum(nblk_e, keepdims=True).astype(jnp.int32)


def _gate(x, gw1, gb1, gw2, gb2):
    return pl.pallas_call(
        _gate_kernel,
        out_shape=[
            jax.ShapeDtypeStruct((2, _T), jnp.float32),
            jax.ShapeDtypeStruct((2, _T), jnp.int32),
            jax.ShapeDtypeStruct((1, _NBLK), jnp.int32),
            jax.ShapeDtypeStruct((1, 1), jnp.int32),
            jax.ShapeDtypeStruct((1, 1), jnp.float32),
        ],
    )(x, gw1, gb1.reshape(1, _G), gw2, gb2.reshape(1, _E))


_NW = 32
_PW = (2 * _T) // _NW
_D2 = _D // 2


def _sc_scatter(x, posF):
    @functools.partial(
        pl.kernel,
        out_type=jax.ShapeDtypeStruct((_NROWS, _D), jnp.float32),
        mesh=plsc.VectorSubcoreMesh(core_axis_name="c", subcore_axis_name="s"),
        scratch_types=[
            pltpu.VMEM((_PW,), jnp.int32),
            pltpu.VMEM((_PW, _D), jnp.float32),
            pltpu.SemaphoreType.DMA,
        ],
    )
    def k(x_hbm, i_hbm, xs_hbm, idx_v, rows_v, sem):
        wid = jax.lax.axis_index("s") * 2 + jax.lax.axis_index("c")
        base = wid * _PW
        tok_base = (wid % (_T // _PW)) * _PW
        pltpu.sync_copy(i_hbm.at[pl.ds(base, _PW)], idx_v)
        pltpu.sync_copy(x_hbm.at[pl.ds(tok_base, _PW)], rows_v)
        pltpu.async_copy(rows_v, xs_hbm.at[idx_v], sem).wait()

    return k(x, posF)


def _sc_gather(ys, posF):
    @functools.partial(
        pl.kernel,
        out_type=jax.ShapeDtypeStruct((2 * _T, _D), jnp.float32),
        mesh=plsc.VectorSubcoreMesh(core_axis_name="c", subcore_axis_name="s"),
        scratch_types=[
            pltpu.VMEM((_PW,), jnp.int32),
            pltpu.VMEM((_PW, _D), jnp.float32),
            pltpu.SemaphoreType.DMA,
        ],
    )
    def k(ys_hbm, i_hbm, o_hbm, idx_v, rows_v, sem):
        wid = jax.lax.axis_index("s") * 2 + jax.lax.axis_index("c")
        base = wid * _PW
        pltpu.sync_copy(i_hbm.at[pl.ds(base, _PW)], idx_v)
        pltpu.async_copy(ys_hbm.at[idx_v], rows_v, sem).wait()
        pltpu.sync_copy(rows_v, o_hbm.at[pl.ds(base, _PW)])

    return k(ys, posF)


def _gemm_kernel(bexp_ref, nact_ref, xs_ref, wh_ref, bh_ref, wo_ref, bo_ref,
                 out_ref):
    b = pl.program_id(0)

    @pl.when(b < nact_ref[0])
    def _():
        h = jnp.maximum(
            jnp.dot(xs_ref[...], wh_ref[0], preferred_element_type=jnp.float32)
            + bh_ref[0], 0.0)
        out_ref[...] = (
            jnp.dot(h, wo_ref[0], preferred_element_type=jnp.float32)
            + bo_ref[0])


def _grouped_gemm(bexp, nact, xs, wh, bh, wo, bo):
    def _clamp(b, bexp_ref, nact_ref):
        return bexp_ref[jnp.minimum(b, nact_ref[0] - 1)]

    grid_spec = pltpu.PrefetchScalarGridSpec(
        num_scalar_prefetch=2,
        grid=(_NBLK,),
        in_specs=[
            pl.BlockSpec((_B, _D),
                         lambda b, be, na: (jnp.minimum(b, na[0] - 1), 0)),
            pl.BlockSpec((1, _D, _H), lambda b, be, na: (_clamp(b, be, na), 0, 0)),
            pl.BlockSpec((1, 1, _H), lambda b, be, na: (_clamp(b, be, na), 0, 0)),
            pl.BlockSpec((1, _H, _D), lambda b, be, na: (_clamp(b, be, na), 0, 0)),
            pl.BlockSpec((1, 1, _D), lambda b, be, na: (_clamp(b, be, na), 0, 0)),
        ],
        out_specs=pl.BlockSpec(
            (_B, _D),
            lambda b, be, na: (jnp.where(b < na[0], b, _NBLK - 1), 0)),
    )
    return pl.pallas_call(
        _gemm_kernel,
        grid_spec=grid_spec,
        out_shape=jax.ShapeDtypeStruct((_NROWS, _D), jnp.float32),
        compiler_params=pltpu.CompilerParams(
            dimension_semantics=("parallel",),
        ),
    )(bexp, nact, xs, wh, bh.reshape(_E, 1, _H), wo, bo.reshape(_E, 1, _D))


_BC = 512


def _combine_kernel(y1_ref, y2_ref, w_ref, out_ref):
    out_ref[...] = (y1_ref[...] * w_ref[:, 0:1] + y2_ref[...] * w_ref[:, 1:2])


def _combine(ysg, w12):
    nt = _T // _BC
    return pl.pallas_call(
        _combine_kernel,
        grid=(nt,),
        in_specs=[
            pl.BlockSpec((_BC, _D), lambda i: (i, 0)),
            pl.BlockSpec((_BC, _D), lambda i: (i + _T // _BC, 0)),
            pl.BlockSpec((_BC, 2), lambda i: (i, 0)),
        ],
        out_specs=pl.BlockSpec((_BC, _D), lambda i: (i, 0)),
        out_shape=jax.ShapeDtypeStruct((_T, _D), jnp.float32),
        compiler_params=pltpu.CompilerParams(
            dimension_semantics=("parallel",),
        ),
    )(ysg, ysg, w12)


def kernel(x, gw1, gb1, gw2, gb2, wh, bh, wo, bo):
    w12T, posT, bexp, nact, aux = _gate(x, gw1, gb1, gw2, gb2)
    w12 = w12T.T
    posF = posT.reshape(2 * _T)
    xs = _sc_scatter(x, posF)
    ys = _grouped_gemm(bexp.reshape(_NBLK), nact.reshape(1), xs,
                       wh, bh, wo, bo)
    ysg = _sc_gather(ys, posF)
    out = _combine(ysg, w12)
    return out, aux.reshape(())


# speedup vs baseline: 1.0113x; 1.0113x over previous
import functools

import jax
import jax.numpy as jnp
from jax.experimental import pallas as pl
from jax.experimental.pallas import tpu as pltpu
from jax.experimental.pallas import tpu_sc as plsc

_T, _D, _G, _H, _E = 2048, 768, 256, 1536, 8
_B = 512
_NBLK = (2 * _T) // _B + _E
_NROWS = _NBLK * _B
_W = 64


def _gate_kernel(x_ref, gw1_ref, gb1_ref, gw2_ref, gb2_ref,
                 w12T_ref, posT_ref, bexp_ref, nact_ref, aux_ref):
    x = x_ref[...]
    gh = jnp.maximum(
        jnp.dot(x, gw1_ref[...], preferred_element_type=jnp.float32)
        + gb1_ref[...], 0.0)
    logits = (jnp.dot(gh, gw2_ref[...], preferred_element_type=jnp.float32)
              + gb2_ref[...])
    lt = logits.T
    m = jnp.max(lt, axis=0, keepdims=True)
    ex = jnp.exp(lt - m)
    p = ex / jnp.sum(ex, axis=0, keepdims=True)

    sub = jax.lax.broadcasted_iota(jnp.int32, (_E, _T), 0)
    m1 = jnp.max(p, axis=0, keepdims=True)
    i1 = jnp.min(jnp.where(p == m1, sub, _E), axis=0, keepdims=True)
    pm = jnp.where(sub == i1, -1.0, p)
    m2 = jnp.max(pm, axis=0, keepdims=True)
    i2 = jnp.min(jnp.where(pm == m2, sub, _E), axis=0, keepdims=True)

    denom = m1 + m2 + 1e-9
    w12T_ref[...] = jnp.concatenate([m1 / denom, m2 / denom], axis=0)

    oh1 = jnp.where(sub == i1, 1.0, 0.0)
    oh2 = jnp.where(sub == i2, 1.0, 0.0)
    c = oh1 + oh2
    cnt = jnp.sum(c, axis=1, keepdims=True)
    load = cnt / (_T + 1e-9)
    lbm = jnp.mean(load)
    lbl = jnp.sum((load - lbm) ** 2) / (_E - 1)
    ent = -jnp.sum(p * jnp.log(p + 1e-9)) / _T
    aux_ref[...] = jnp.reshape(lbl + ent, (1, 1))

    _CH = 128
    _NC = _T // _CH
    r_i = jax.lax.broadcasted_iota(jnp.int32, (_CH, _CH), 0)
    c_i = jax.lax.broadcasted_iota(jnp.int32, (_CH, _CH), 1)
    ustri = jnp.where(r_i < c_i, 1.0, 0.0)
    parts = []
    sums = []
    for k in range(_NC):
        ck = c[:, k * _CH:(k + 1) * _CH]
        parts.append(jnp.dot(ck, ustri, preferred_element_type=jnp.float32))
        sums.append(jnp.sum(ck, axis=1, keepdims=True))
    s = jnp.concatenate(sums, axis=1)
    kr_i = jax.lax.broadcasted_iota(jnp.int32, (_NC, _NC), 0)
    kc_i = jax.lax.broadcasted_iota(jnp.int32, (_NC, _NC), 1)
    kustri = jnp.where(kr_i < kc_i, 1.0, 0.0)
    pref = jnp.dot(s, kustri, preferred_element_type=jnp.float32)
    rank = jnp.concatenate(
        [parts[k] + pref[:, k:k + 1] for k in range(_NC)], axis=1)

    nblk_e = jnp.floor((cnt + (_B - 1)) / _B)
    e_r = jax.lax.broadcasted_iota(jnp.int32, (_E, _E), 0)
    e_c = jax.lax.broadcasted_iota(jnp.int32, (_E, _E), 1)
    ltri_inc = jnp.where(e_c <= e_r, 1.0, 0.0)
    ends = jnp.dot(ltri_inc, nblk_e,
                   preferred_element_type=jnp.float32)
    starts = ends - nblk_e
    rowoff = starts * _B

    posm = rowoff + rank
    pos1 = jnp.sum(jnp.where(sub == i1, posm, 0.0), axis=0, keepdims=True)
    pos2 = jnp.sum(jnp.where(sub == i2, posm, 0.0), axis=0, keepdims=True)
    posT_ref[...] = jnp.concatenate([pos1, pos2], axis=0).astype(jnp.int32)

    b_iota = jax.lax.broadcasted_iota(jnp.int32, (1, _NBLK), 1).astype(
        jnp.float32)
    bexp = jnp.zeros((1, _NBLK), jnp.float32)
    for e in range(_E):
        bexp = bexp + jnp.where(b_iota >= ends[e:e + 1, 0:1], 1.0, 0.0)
    bexp_ref[...] = jnp.minimum(bexp, _E - 1).astype(jnp.int32)
    nact_ref[...] = jnp.sum(nblk_e, keepdims=True).astype(jnp.int32)


def _gate(x, gw1, gb1, gw2, gb2):
    return pl.pallas_call(
        _gate_kernel,
        out_shape=[
            jax.ShapeDtypeStruct((2, _T), jnp.float32),
            jax.ShapeDtypeStruct((2, _T), jnp.int32),
            jax.ShapeDtypeStruct((1, _NBLK), jnp.int32),
            jax.ShapeDtypeStruct((1, 1), jnp.int32),
            jax.ShapeDtypeStruct((1, 1), jnp.float32),
        ],
    )(x, gw1, gb1.reshape(1, _G), gw2, gb2.reshape(1, _E))


_NW = 32
_PW = (2 * _T) // _NW
_D2 = _D // 2


def _sc_scatter(x, posF):
    @functools.partial(
        pl.kernel,
        out_type=jax.ShapeDtypeStruct((_NROWS, _D), jnp.float32),
        mesh=plsc.VectorSubcoreMesh(core_axis_name="c", subcore_axis_name="s"),
        scratch_types=[
            pltpu.VMEM((_PW,), jnp.int32),
            pltpu.VMEM((_PW, _D), jnp.float32),
            pltpu.SemaphoreType.DMA,
        ],
    )
    def k(x_hbm, i_hbm, xs_hbm, idx_v, rows_v, sem):
        wid = jax.lax.axis_index("s") * 2 + jax.lax.axis_index("c")
        slot = wid // (_T // _PW)
        tok_base = (wid % (_T // _PW)) * _PW
        pltpu.sync_copy(i_hbm.at[slot, pl.ds(tok_base, _PW)], idx_v)
        pltpu.sync_copy(x_hbm.at[pl.ds(tok_base, _PW)], rows_v)
        pltpu.async_copy(rows_v, xs_hbm.at[idx_v], sem).wait()

    return k(x, posF)


def _sc_gather(ys, posF):
    @functools.partial(
        pl.kernel,
        out_type=jax.ShapeDtypeStruct((2 * _T, _D), jnp.float32),
        mesh=plsc.VectorSubcoreMesh(core_axis_name="c", subcore_axis_name="s"),
        scratch_types=[
            pltpu.VMEM((_PW,), jnp.int32),
            pltpu.VMEM((_PW, _D), jnp.float32),
            pltpu.SemaphoreType.DMA,
        ],
    )
    def k(ys_hbm, i_hbm, o_hbm, idx_v, rows_v, sem):
        wid = jax.lax.axis_index("s") * 2 + jax.lax.axis_index("c")
        base = wid * _PW
        slot = wid // (_T // _PW)
        col = (wid % (_T // _PW)) * _PW
        pltpu.sync_copy(i_hbm.at[slot, pl.ds(col, _PW)], idx_v)
        pltpu.async_copy(ys_hbm.at[idx_v], rows_v, sem).wait()
        pltpu.sync_copy(rows_v, o_hbm.at[pl.ds(base, _PW)])

    return k(ys, posF)


def _gemm_kernel(bexp_ref, nact_ref, xs_ref, wh_ref, bh_ref, wo_ref, bo_ref,
                 out_ref):
    b = pl.program_id(0)

    @pl.when(b < nact_ref[0])
    def _():
        h = jnp.maximum(
            jnp.dot(xs_ref[...], wh_ref[0], preferred_element_type=jnp.float32)
            + bh_ref[0], 0.0)
        out_ref[...] = (
            jnp.dot(h, wo_ref[0], preferred_element_type=jnp.float32)
            + bo_ref[0])


def _grouped_gemm(bexp, nact, xs, wh, bh, wo, bo):
    def _clamp(b, bexp_ref, nact_ref):
        return bexp_ref[jnp.minimum(b, nact_ref[0] - 1)]

    grid_spec = pltpu.PrefetchScalarGridSpec(
        num_scalar_prefetch=2,
        grid=(_NBLK,),
        in_specs=[
            pl.BlockSpec((_B, _D),
                         lambda b, be, na: (jnp.minimum(b, na[0] - 1), 0)),
            pl.BlockSpec((1, _D, _H), lambda b, be, na: (_clamp(b, be, na), 0, 0)),
            pl.BlockSpec((1, 1, _H), lambda b, be, na: (_clamp(b, be, na), 0, 0)),
            pl.BlockSpec((1, _H, _D), lambda b, be, na: (_clamp(b, be, na), 0, 0)),
            pl.BlockSpec((1, 1, _D), lambda b, be, na: (_clamp(b, be, na), 0, 0)),
        ],
        out_specs=pl.BlockSpec(
            (_B, _D),
            lambda b, be, na: (jnp.where(b < na[0], b, _NBLK - 1), 0)),
    )
    return pl.pallas_call(
        _gemm_kernel,
        grid_spec=grid_spec,
        out_shape=jax.ShapeDtypeStruct((_NROWS, _D), jnp.float32),
        compiler_params=pltpu.CompilerParams(
            dimension_semantics=("parallel",),
        ),
    )(bexp, nact, xs, wh, bh.reshape(_E, 1, _H), wo, bo.reshape(_E, 1, _D))


_BC = 512


def _combine_kernel(y1_ref, y2_ref, w_ref, out_ref):
    out_ref[...] = (y1_ref[...] * w_ref[:, 0:1] + y2_ref[...] * w_ref[:, 1:2])


def _combine(ysg, w12):
    nt = _T // _BC
    return pl.pallas_call(
        _combine_kernel,
        grid=(nt,),
        in_specs=[
            pl.BlockSpec((_BC, _D), lambda i: (i, 0)),
            pl.BlockSpec((_BC, _D), lambda i: (i + _T // _BC, 0)),
            pl.BlockSpec((_BC, 2), lambda i: (i, 0)),
        ],
        out_specs=pl.BlockSpec((_BC, _D), lambda i: (i, 0)),
        out_shape=jax.ShapeDtypeStruct((_T, _D), jnp.float32),
        compiler_params=pltpu.CompilerParams(
            dimension_semantics=("parallel",),
        ),
    )(ysg, ysg, w12)


def kernel(x, gw1, gb1, gw2, gb2, wh, bh, wo, bo):
    w12T, posT, bexp, nact, aux = _gate(x, gw1, gb1, gw2, gb2)
    w12 = w12T.T
    posF = posT
    xs = _sc_scatter(x, posF)
    ys = _grouped_gemm(bexp.reshape(_NBLK), nact.reshape(1), xs,
                       wh, bh, wo, bo)
    ysg = _sc_gather(ys, posF)
    out = _combine(ysg, w12)
    return out, aux.reshape(())


# speedup vs baseline: 1.0113x; 1.0001x over previous
import functools

import jax
import jax.numpy as jnp
from jax.experimental import pallas as pl
from jax.experimental.pallas import tpu as pltpu
from jax.experimental.pallas import tpu_sc as plsc

_T, _D, _G, _H, _E = 2048, 768, 256, 1536, 8
_B = 512
_NBLK = (2 * _T) // _B + _E
_NROWS = _NBLK * _B
_W = 64


def _route_kernel(x_ref, gw1_ref, gb1_ref, gw2_ref, gb2_ref,
                  logits_ref, posT_ref, bexp_ref, nact_ref, cnt_ref):
    x = x_ref[...]
    gh = jnp.maximum(
        jnp.dot(x, gw1_ref[...], preferred_element_type=jnp.float32)
        + gb1_ref[...], 0.0)
    logits = (jnp.dot(gh, gw2_ref[...], preferred_element_type=jnp.float32)
              + gb2_ref[...])
    logits_ref[...] = logits
    lt = logits.T
    sub = jax.lax.broadcasted_iota(jnp.int32, (_E, _T), 0)
    m1 = jnp.max(lt, axis=0, keepdims=True)
    i1 = jnp.min(jnp.where(lt == m1, sub, _E), axis=0, keepdims=True)
    pm = jnp.where(sub == i1, -jnp.inf, lt)
    m2 = jnp.max(pm, axis=0, keepdims=True)
    i2 = jnp.min(jnp.where(pm == m2, sub, _E), axis=0, keepdims=True)

    c = (jnp.where(sub == i1, 1.0, 0.0)
         + jnp.where(sub == i2, 1.0, 0.0))
    cnt = jnp.sum(c, axis=1, keepdims=True)
    cnt_ref[...] = cnt

    _CH = 128
    _NC = _T // _CH
    r_i = jax.lax.broadcasted_iota(jnp.int32, (_CH, _CH), 0)
    c_i = jax.lax.broadcasted_iota(jnp.int32, (_CH, _CH), 1)
    ustri = jnp.where(r_i < c_i, 1.0, 0.0)
    parts = []
    sums = []
    for k in range(_NC):
        ck = c[:, k * _CH:(k + 1) * _CH]
        parts.append(jnp.dot(ck, ustri, preferred_element_type=jnp.float32))
        sums.append(jnp.sum(ck, axis=1, keepdims=True))
    s = jnp.concatenate(sums, axis=1)
    kr_i = jax.lax.broadcasted_iota(jnp.int32, (_NC, _NC), 0)
    kc_i = jax.lax.broadcasted_iota(jnp.int32, (_NC, _NC), 1)
    kustri = jnp.where(kr_i < kc_i, 1.0, 0.0)
    pref = jnp.dot(s, kustri, preferred_element_type=jnp.float32)
    rank = jnp.concatenate(
        [parts[k] + pref[:, k:k + 1] for k in range(_NC)], axis=1)

    nblk_e = jnp.floor((cnt + (_B - 1)) / _B)
    e_r = jax.lax.broadcasted_iota(jnp.int32, (_E, _E), 0)
    e_c = jax.lax.broadcasted_iota(jnp.int32, (_E, _E), 1)
    ltri_inc = jnp.where(e_c <= e_r, 1.0, 0.0)
    ends = jnp.dot(ltri_inc, nblk_e,
                   preferred_element_type=jnp.float32)
    rowoff = (ends - nblk_e) * _B

    posm = rowoff + rank
    pos1 = jnp.sum(jnp.where(sub == i1, posm, 0.0), axis=0, keepdims=True)
    pos2 = jnp.sum(jnp.where(sub == i2, posm, 0.0), axis=0, keepdims=True)
    posT_ref[...] = jnp.concatenate([pos1, pos2], axis=0).astype(jnp.int32)

    b_iota = jax.lax.broadcasted_iota(jnp.int32, (1, _NBLK), 1).astype(
        jnp.float32)
    bexp = jnp.zeros((1, _NBLK), jnp.float32)
    for e in range(_E):
        bexp = bexp + jnp.where(b_iota >= ends[e:e + 1, 0:1], 1.0, 0.0)
    bexp_ref[...] = jnp.minimum(bexp, _E - 1).astype(jnp.int32)
    nact_ref[...] = jnp.sum(nblk_e, keepdims=True).astype(jnp.int32)


def _route(x, gw1, gb1, gw2, gb2):
    return pl.pallas_call(
        _route_kernel,
        out_shape=[
            jax.ShapeDtypeStruct((_T, _E), jnp.float32),
            jax.ShapeDtypeStruct((2, _T), jnp.int32),
            jax.ShapeDtypeStruct((1, _NBLK), jnp.int32),
            jax.ShapeDtypeStruct((1, 1), jnp.int32),
            jax.ShapeDtypeStruct((_E, 1), jnp.float32),
        ],
    )(x, gw1, gb1.reshape(1, _G), gw2, gb2.reshape(1, _E))


def _aux_kernel(logits_ref, cnt_ref, w12T_ref, aux_ref):
    lt = logits_ref[...].T
    m = jnp.max(lt, axis=0, keepdims=True)
    ex = jnp.exp(lt - m)
    p = ex / jnp.sum(ex, axis=0, keepdims=True)
    sub = jax.lax.broadcasted_iota(jnp.int32, (_E, _T), 0)
    m1 = jnp.max(p, axis=0, keepdims=True)
    i1 = jnp.min(jnp.where(p == m1, sub, _E), axis=0, keepdims=True)
    pm = jnp.where(sub == i1, -1.0, p)
    m2 = jnp.max(pm, axis=0, keepdims=True)
    denom = m1 + m2 + 1e-9
    w12T_ref[...] = jnp.concatenate([m1 / denom, m2 / denom], axis=0)
    load = cnt_ref[...] / (_T + 1e-9)
    lbm = jnp.mean(load)
    lbl = jnp.sum((load - lbm) ** 2) / (_E - 1)
    ent = -jnp.sum(p * jnp.log(p + 1e-9)) / _T
    aux_ref[...] = jnp.reshape(lbl + ent, (1, 1))


def _aux(logits, cnt):
    return pl.pallas_call(
        _aux_kernel,
        out_shape=[
            jax.ShapeDtypeStruct((2, _T), jnp.float32),
            jax.ShapeDtypeStruct((1, 1), jnp.float32),
        ],
    )(logits, cnt)


_NW = 32
_PW = (2 * _T) // _NW
_D2 = _D // 2


def _sc_scatter(x, posF):
    @functools.partial(
        pl.kernel,
        out_type=jax.ShapeDtypeStruct((_NROWS, _D), jnp.float32),
        mesh=plsc.VectorSubcoreMesh(core_axis_name="c", subcore_axis_name="s"),
        scratch_types=[
            pltpu.VMEM((_PW,), jnp.int32),
            pltpu.VMEM((_PW, _D), jnp.float32),
            pltpu.SemaphoreType.DMA,
        ],
    )
    def k(x_hbm, i_hbm, xs_hbm, idx_v, rows_v, sem):
        wid = jax.lax.axis_index("s") * 2 + jax.lax.axis_index("c")
        slot = wid // (_T // _PW)
        tok_base = (wid % (_T // _PW)) * _PW
        pltpu.sync_copy(i_hbm.at[slot, pl.ds(tok_base, _PW)], idx_v)
        pltpu.sync_copy(x_hbm.at[pl.ds(tok_base, _PW)], rows_v)
        pltpu.async_copy(rows_v, xs_hbm.at[idx_v], sem).wait()

    return k(x, posF)


def _sc_gather(ys, posF):
    @functools.partial(
        pl.kernel,
        out_type=jax.ShapeDtypeStruct((2 * _T, _D), jnp.float32),
        mesh=plsc.VectorSubcoreMesh(core_axis_name="c", subcore_axis_name="s"),
        scratch_types=[
            pltpu.VMEM((_PW,), jnp.int32),
            pltpu.VMEM((_PW, _D), jnp.float32),
            pltpu.SemaphoreType.DMA,
        ],
    )
    def k(ys_hbm, i_hbm, o_hbm, idx_v, rows_v, sem):
        wid = jax.lax.axis_index("s") * 2 + jax.lax.axis_index("c")
        base = wid * _PW
        slot = wid // (_T // _PW)
        col = (wid % (_T // _PW)) * _PW
        pltpu.sync_copy(i_hbm.at[slot, pl.ds(col, _PW)], idx_v)
        pltpu.async_copy(ys_hbm.at[idx_v], rows_v, sem).wait()
        pltpu.sync_copy(rows_v, o_hbm.at[pl.ds(base, _PW)])

    return k(ys, posF)


def _gemm_kernel(bexp_ref, nact_ref, xs_ref, wh_ref, bh_ref, wo_ref, bo_ref,
                 out_ref):
    b = pl.program_id(0)

    @pl.when(b < nact_ref[0])
    def _():
        h = jnp.maximum(
            jnp.dot(xs_ref[...], wh_ref[0], preferred_element_type=jnp.float32)
            + bh_ref[0], 0.0)
        out_ref[...] = (
            jnp.dot(h, wo_ref[0], preferred_element_type=jnp.float32)
            + bo_ref[0])


def _grouped_gemm(bexp, nact, xs, wh, bh, wo, bo):
    def _clamp(b, bexp_ref, nact_ref):
        return bexp_ref[jnp.minimum(b, nact_ref[0] - 1)]

    grid_spec = pltpu.PrefetchScalarGridSpec(
        num_scalar_prefetch=2,
        grid=(_NBLK,),
        in_specs=[
            pl.BlockSpec((_B, _D),
                         lambda b, be, na: (jnp.minimum(b, na[0] - 1), 0)),
            pl.BlockSpec((1, _D, _H), lambda b, be, na: (_clamp(b, be, na), 0, 0)),
            pl.BlockSpec((1, 1, _H), lambda b, be, na: (_clamp(b, be, na), 0, 0)),
            pl.BlockSpec((1, _H, _D), lambda b, be, na: (_clamp(b, be, na), 0, 0)),
            pl.BlockSpec((1, 1, _D), lambda b, be, na: (_clamp(b, be, na), 0, 0)),
        ],
        out_specs=pl.BlockSpec(
            (_B, _D),
            lambda b, be, na: (jnp.where(b < na[0], b, _NBLK - 1), 0)),
    )
    return pl.pallas_call(
        _gemm_kernel,
        grid_spec=grid_spec,
        out_shape=jax.ShapeDtypeStruct((_NROWS, _D), jnp.float32),
        compiler_params=pltpu.CompilerParams(
            dimension_semantics=("parallel",),
        ),
    )(bexp, nact, xs, wh, bh.reshape(_E, 1, _H), wo, bo.reshape(_E, 1, _D))


_BC = 512


def _combine_kernel(y_ref, w_ref, out_ref):
    out_ref[...] = (y_ref[0] * w_ref[:, 0:1] + y_ref[1] * w_ref[:, 1:2])


def _combine(ysg3, w12):
    nt = _T // _BC
    return pl.pallas_call(
        _combine_kernel,
        grid=(nt,),
        in_specs=[
            pl.BlockSpec((2, _BC, _D), lambda i: (0, i, 0)),
            pl.BlockSpec((_BC, 2), lambda i: (i, 0)),
        ],
        out_specs=pl.BlockSpec((_BC, _D), lambda i: (i, 0)),
        out_shape=jax.ShapeDtypeStruct((_T, _D), jnp.float32),
        compiler_params=pltpu.CompilerParams(
            dimension_semantics=("parallel",),
        ),
    )(ysg3, w12)


def kernel(x, gw1, gb1, gw2, gb2, wh, bh, wo, bo):
    logits, posT, bexp, nact, cnt = _route(x, gw1, gb1, gw2, gb2)
    w12T, aux = _aux(logits, cnt)
    xs = _sc_scatter(x, posT)
    ys = _grouped_gemm(bexp.reshape(_NBLK), nact.reshape(1), xs,
                       wh, bh, wo, bo)
    ysg = _sc_gather(ys, posT)
    out = _combine(ysg.reshape(2, _T, _D), w12T.T)
    return out, aux.reshape(())


# speedup vs baseline: 1.0116x; 1.0003x over previous
import functools

import jax
import jax.numpy as jnp
from jax.experimental import pallas as pl
from jax.experimental.pallas import tpu as pltpu
from jax.experimental.pallas import tpu_sc as plsc

_T, _D, _G, _H, _E = 2048, 768, 256, 1536, 8
_B = 512
_NBLK = (2 * _T) // _B + _E
_NROWS = _NBLK * _B


def _route_kernel(x_ref, gw1_ref, gb1_ref, gw2_ref, gb2_ref,
                  logits_ref, posT_ref, bexp_ref, nact_ref, cnt_ref):
    x = x_ref[...]
    gh = jnp.maximum(
        jnp.dot(x, gw1_ref[...], preferred_element_type=jnp.float32)
        + gb1_ref[...], 0.0)
    logits = (jnp.dot(gh, gw2_ref[...], preferred_element_type=jnp.float32)
              + gb2_ref[...])
    logits_ref[...] = logits
    lt = logits.T
    sub = jax.lax.broadcasted_iota(jnp.int32, (_E, _T), 0)
    m1 = jnp.max(lt, axis=0, keepdims=True)
    i1 = jnp.min(jnp.where(lt == m1, sub, _E), axis=0, keepdims=True)
    pm = jnp.where(sub == i1, -jnp.inf, lt)
    m2 = jnp.max(pm, axis=0, keepdims=True)
    i2 = jnp.min(jnp.where(pm == m2, sub, _E), axis=0, keepdims=True)

    c = (jnp.where(sub == i1, 1.0, 0.0)
         + jnp.where(sub == i2, 1.0, 0.0))
    cnt = jnp.sum(c, axis=1, keepdims=True)
    cnt_ref[...] = cnt

    _CH = 128
    _NC = _T // _CH
    r_i = jax.lax.broadcasted_iota(jnp.int32, (_CH, _CH), 0)
    c_i = jax.lax.broadcasted_iota(jnp.int32, (_CH, _CH), 1)
    ustri = jnp.where(r_i < c_i, 1.0, 0.0)
    parts = []
    sums = []
    for k in range(_NC):
        ck = c[:, k * _CH:(k + 1) * _CH]
        parts.append(jnp.dot(ck, ustri, preferred_element_type=jnp.float32))
        sums.append(jnp.sum(ck, axis=1, keepdims=True))
    s = jnp.concatenate(sums, axis=1)
    kr_i = jax.lax.broadcasted_iota(jnp.int32, (_NC, _NC), 0)
    kc_i = jax.lax.broadcasted_iota(jnp.int32, (_NC, _NC), 1)
    kustri = jnp.where(kr_i < kc_i, 1.0, 0.0)
    pref = jnp.dot(s, kustri, preferred_element_type=jnp.float32)
    rank = jnp.concatenate(
        [parts[k] + pref[:, k:k + 1] for k in range(_NC)], axis=1)

    nblk_e = jnp.floor((cnt + (_B - 1)) / _B)
    e_r = jax.lax.broadcasted_iota(jnp.int32, (_E, _E), 0)
    e_c = jax.lax.broadcasted_iota(jnp.int32, (_E, _E), 1)
    ltri_inc = jnp.where(e_c <= e_r, 1.0, 0.0)
    ends = jnp.dot(ltri_inc, nblk_e,
                   preferred_element_type=jnp.float32)
    rowoff = (ends - nblk_e) * _B

    posm = rowoff + rank
    pos1 = jnp.sum(jnp.where(sub == i1, posm, 0.0), axis=0, keepdims=True)
    pos2 = jnp.sum(jnp.where(sub == i2, posm, 0.0), axis=0, keepdims=True)
    posT_ref[...] = jnp.concatenate([pos1, pos2], axis=0).astype(jnp.int32)

    b_iota = jax.lax.broadcasted_iota(jnp.int32, (1, _NBLK), 1).astype(
        jnp.float32)
    bexp = jnp.zeros((1, _NBLK), jnp.float32)
    for e in range(_E):
        bexp = bexp + jnp.where(b_iota >= ends[e:e + 1, 0:1], 1.0, 0.0)
    bexp_ref[...] = jnp.minimum(bexp, _E - 1).astype(jnp.int32)
    nact_ref[...] = jnp.sum(nblk_e, keepdims=True).astype(jnp.int32)


def _route(x, gw1, gb1, gw2, gb2):
    return pl.pallas_call(
        _route_kernel,
        out_shape=[
            jax.ShapeDtypeStruct((_T, _E), jnp.float32),
            jax.ShapeDtypeStruct((2, _T), jnp.int32),
            jax.ShapeDtypeStruct((1, _NBLK), jnp.int32),
            jax.ShapeDtypeStruct((1, 1), jnp.int32),
            jax.ShapeDtypeStruct((_E, 1), jnp.float32),
        ],
    )(x, gw1, gb1.reshape(1, _G), gw2, gb2.reshape(1, _E))


def _aux_kernel(logits_ref, cnt_ref, w12T_ref, aux_ref):
    lt = logits_ref[...].T
    m = jnp.max(lt, axis=0, keepdims=True)
    ex = jnp.exp(lt - m)
    p = ex / jnp.sum(ex, axis=0, keepdims=True)
    sub = jax.lax.broadcasted_iota(jnp.int32, (_E, _T), 0)
    m1 = jnp.max(p, axis=0, keepdims=True)
    i1 = jnp.min(jnp.where(p == m1, sub, _E), axis=0, keepdims=True)
    pm = jnp.where(sub == i1, -1.0, p)
    m2 = jnp.max(pm, axis=0, keepdims=True)
    denom = m1 + m2 + 1e-9
    w12T_ref[...] = jnp.concatenate([m1 / denom, m2 / denom], axis=0)
    load = cnt_ref[...] / (_T + 1e-9)
    lbm = jnp.mean(load)
    lbl = jnp.sum((load - lbm) ** 2) / (_E - 1)
    ent = -jnp.sum(p * jnp.log(p + 1e-9)) / _T
    aux_ref[...] = jnp.reshape(lbl + ent, (1, 1))


def _aux(logits, cnt):
    return pl.pallas_call(
        _aux_kernel,
        out_shape=[
            jax.ShapeDtypeStruct((2, _T), jnp.float32),
            jax.ShapeDtypeStruct((1, 1), jnp.float32),
        ],
    )(logits, cnt)


_NW = 32
_PW = (2 * _T) // _NW


def _sc_scatter(x, posF):
    @functools.partial(
        pl.kernel,
        out_type=jax.ShapeDtypeStruct((_NROWS, _D), jnp.float32),
        mesh=plsc.VectorSubcoreMesh(core_axis_name="c", subcore_axis_name="s"),
        scratch_types=[
            pltpu.VMEM((_PW,), jnp.int32),
            pltpu.VMEM((_PW, _D), jnp.float32),
            pltpu.SemaphoreType.DMA,
        ],
    )
    def k(x_hbm, i_hbm, xs_hbm, idx_v, rows_v, sem):
        wid = jax.lax.axis_index("s") * 2 + jax.lax.axis_index("c")
        slot = wid // (_T // _PW)
        tok_base = (wid % (_T // _PW)) * _PW
        pltpu.sync_copy(i_hbm.at[slot, pl.ds(tok_base, _PW)], idx_v)
        pltpu.sync_copy(x_hbm.at[pl.ds(tok_base, _PW)], rows_v)
        pltpu.async_copy(rows_v, xs_hbm.at[idx_v], sem).wait()

    return k(x, posF)


def _sc_gather(ys, posF):
    @functools.partial(
        pl.kernel,
        out_type=jax.ShapeDtypeStruct((2 * _T, _D), jnp.float32),
        mesh=plsc.VectorSubcoreMesh(core_axis_name="c", subcore_axis_name="s"),
        scratch_types=[
            pltpu.VMEM((_PW,), jnp.int32),
            pltpu.VMEM((_PW, _D), jnp.float32),
            pltpu.SemaphoreType.DMA,
        ],
    )
    def k(ys_hbm, i_hbm, o_hbm, idx_v, rows_v, sem):
        wid = jax.lax.axis_index("s") * 2 + jax.lax.axis_index("c")
        base = wid * _PW
        slot = wid // (_T // _PW)
        col = (wid % (_T // _PW)) * _PW
        pltpu.sync_copy(i_hbm.at[slot, pl.ds(col, _PW)], idx_v)
        pltpu.async_copy(ys_hbm.at[idx_v], rows_v, sem).wait()
        pltpu.sync_copy(rows_v, o_hbm.at[pl.ds(base, _PW)])

    return k(ys, posF)


def _gemm_kernel(bexp_ref, nact_ref, xs_ref, wh_ref, bh_ref, wo_ref, bo_ref,
                 out_ref):
    b = pl.program_id(0)

    @pl.when(b < nact_ref[0])
    def _():
        h = jnp.maximum(
            jnp.dot(xs_ref[...], wh_ref[0], preferred_element_type=jnp.float32)
            + bh_ref[0], 0.0)
        out_ref[...] = (
            jnp.dot(h, wo_ref[0], preferred_element_type=jnp.float32)
            + bo_ref[0])


def _grouped_gemm(bexp, nact, xs, wh, bh, wo, bo):
    def _clamp(b, bexp_ref, nact_ref):
        return bexp_ref[jnp.minimum(b, nact_ref[0] - 1)]

    grid_spec = pltpu.PrefetchScalarGridSpec(
        num_scalar_prefetch=2,
        grid=(_NBLK,),
        in_specs=[
            pl.BlockSpec((_B, _D),
                         lambda b, be, na: (jnp.minimum(b, na[0] - 1), 0)),
            pl.BlockSpec((1, _D, _H), lambda b, be, na: (_clamp(b, be, na), 0, 0)),
            pl.BlockSpec((1, 1, _H), lambda b, be, na: (_clamp(b, be, na), 0, 0)),
            pl.BlockSpec((1, _H, _D), lambda b, be, na: (_clamp(b, be, na), 0, 0)),
            pl.BlockSpec((1, 1, _D), lambda b, be, na: (_clamp(b, be, na), 0, 0)),
        ],
        out_specs=pl.BlockSpec(
            (_B, _D),
            lambda b, be, na: (jnp.where(b < na[0], b, _NBLK - 1), 0)),
    )
    return pl.pallas_call(
        _gemm_kernel,
        grid_spec=grid_spec,
        out_shape=jax.ShapeDtypeStruct((_NROWS, _D), jnp.float32),
        compiler_params=pltpu.CompilerParams(
            dimension_semantics=("parallel",),
        ),
    )(bexp, nact, xs, wh, bh.reshape(_E, 1, _H), wo, bo.reshape(_E, 1, _D))


_BC = 512


def _combine_kernel(y_ref, w_ref, out_ref):
    out_ref[...] = (y_ref[0] * w_ref[:, 0:1] + y_ref[1] * w_ref[:, 1:2])


def _combine(ysg3, w12):
    nt = _T // _BC
    return pl.pallas_call(
        _combine_kernel,
        grid=(nt,),
        in_specs=[
            pl.BlockSpec((2, _BC, _D), lambda i: (0, i, 0)),
            pl.BlockSpec((_BC, 2), lambda i: (i, 0)),
        ],
        out_specs=pl.BlockSpec((_BC, _D), lambda i: (i, 0)),
        out_shape=jax.ShapeDtypeStruct((_T, _D), jnp.float32),
        compiler_params=pltpu.CompilerParams(
            dimension_semantics=("parallel",),
        ),
    )(ysg3, w12)


def kernel(x, gw1, gb1, gw2, gb2, wh, bh, wo, bo):
    logits, posT, bexp, nact, cnt = _route(x, gw1, gb1, gw2, gb2)
    w12T, aux = _aux(logits, cnt)
    xs = _sc_scatter(x, posT)
    ys = _grouped_gemm(bexp.reshape(_NBLK), nact.reshape(1), xs,
                       wh, bh, wo, bo)
    ysg = _sc_gather(ys, posT)
    out = _combine(ysg.reshape(2, _T, _D), w12T.T)
    return out, aux.reshape(())
